# baseline (speedup 1.0000x reference)
import jax
import jax.numpy as jnp
from jax import lax
from jax.experimental import pallas as pl
from jax.experimental.pallas import tpu as pltpu


def _make_kernel(H, W, Cin, C1, C3, C0, C2, C4, B, PAD, n_steps):
    P = H * W
    Pp = P + 2 * PAD
    R0 = 2 * Cin
    R3B = 3 * C1
    R5B = 5 * C3
    O5 = R0 + 3 * R3B

    def compute_chunk(slot, x_buf, o_buf, wb_ref, wm_ref, bb_ref, bo_ref,
                      u_ref):
        lane = lax.broadcasted_iota(jnp.int32, (1, P), 1)
        wi = lane % W
        hi = lane // W
        def wshift(a, d, fill):
            if d == 0:
                return a
            m = (wi + d >= 0) & (wi + d < W)
            return jnp.where(m, pltpu.roll(a, (-d) % P, axis=1), fill)

        def hshift(a, d, fill):
            m = (hi + d >= 0) & (hi + d < H)
            return jnp.where(m, pltpu.roll(a, (-d * W) % P, axis=1), fill)

        for b in range(B):
            xb = x_buf[slot, b].astype(jnp.bfloat16)

            t = jnp.dot(wb_ref[...], xb, preferred_element_type=jnp.float32)
            t = (t + bb_ref[...]).astype(jnp.bfloat16)
            tp = jnp.pad(t, ((0, 0), (PAD, PAD)))
            tp3 = tp[:C1]
            tp5 = tp[C1:]

            neg = jnp.array(-jnp.inf, jnp.bfloat16)
            m1 = jnp.maximum(jnp.maximum(wshift(xb, -1, neg),
                                         wshift(xb, 1, neg)), xb)
            mp = jnp.maximum(jnp.maximum(hshift(m1, -1, neg),
                                         hshift(m1, 1, neg)), m1)

            p = b % 2
            u_ref[p, 0:Cin] = xb
            u_ref[p, Cin:R0] = mp
            zero = jnp.zeros((), jnp.bfloat16)
            for j, dh in enumerate((-1, 0, 1)):
                for i, dw in enumerate((-1, 0, 1)):
                    m = (wi + dw >= 0) & (wi + dw < W)
                    r = pltpu.roll(tp3, (-(dh * W + dw)) % Pp,
                                   axis=1)[:, PAD:PAD + P]
                    u_ref[p, R0 + C1 * (3 * j + i):
                          R0 + C1 * (3 * j + i + 1)] = jnp.where(m, r, zero)
            for j, dh in enumerate((-2, -1, 0, 1, 2)):
                for i, dw in enumerate((-2, -1, 0, 1, 2)):
                    m = (wi + dw >= 0) & (wi + dw < W)
                    r = pltpu.roll(tp5, (-(dh * W + dw)) % Pp,
                                   axis=1)[:, PAD:PAD + P]
                    u_ref[p, O5 + C3 * (5 * j + i):
                          O5 + C3 * (5 * j + i + 1)] = jnp.where(m, r, zero)

            acc = jnp.dot(wm_ref[...], u_ref[p],
                          preferred_element_type=jnp.float32)
            o_buf[slot, b] = acc + bo_ref[...]

    def body(x_hbm, wb_ref, wm_ref, bb_ref, bo_ref, o_hbm,
             x_buf, o_buf, u_ref, in_sem, out_sem):

        def dma_in(slot, step):
            pltpu.make_async_copy(x_hbm.at[pl.ds(step * B, B)],
                                  x_buf.at[slot], in_sem.at[slot]).start()

        def wait_in(slot):
            pltpu.make_async_copy(x_hbm.at[pl.ds(0, B)],
                                  x_buf.at[slot], in_sem.at[slot]).wait()

        def dma_out(slot, step):
            for b in range(B):
                pltpu.make_async_copy(o_buf.at[slot, b],
                                      o_hbm.at[step * B + b],
                                      out_sem.at[slot]).start()

        def wait_out(slot):
            for b in range(B):
                pltpu.make_async_copy(o_buf.at[slot, 0],
                                      o_hbm.at[0],
                                      out_sem.at[slot]).wait()

        dma_in(0, 0)

        def step_body(step, _):
            cur = lax.rem(step, 2)
            nxt = lax.rem(step + 1, 2)

            @pl.when(step + 1 < n_steps)
            def _():
                dma_in(nxt, step + 1)

            wait_in(cur)

            @pl.when(step >= 2)
            def _():
                wait_out(cur)

            compute_chunk(cur, x_buf, o_buf, wb_ref, wm_ref, bb_ref, bo_ref,
                          u_ref)
            dma_out(cur, step)
            return ()

        lax.fori_loop(0, n_steps, step_body, ())
        if n_steps >= 2:
            wait_out(jnp.int32((n_steps - 2) % 2))
        wait_out(jnp.int32((n_steps - 1) % 2))

    return body


def kernel(x, w1, b1, w3a, b3a, w3b, b3b, w5a, b5a, w5b, b5b, wp, bp):
    N, Cin, H, W = x.shape
    C0 = w1.shape[1]
    C1, C2 = w3b.shape[2], w3b.shape[3]
    C3, C4 = w5b.shape[2], w5b.shape[3]
    C5 = wp.shape[1]
    Ctot = C0 + C2 + C4 + C5
    P = H * W
    PAD = 128
    R0 = 2 * Cin

    f32, bf16 = jnp.float32, jnp.bfloat16

    wb = jnp.concatenate([w3a.T, w5a.T], axis=0).astype(bf16)
    bb = jnp.concatenate([b3a.reshape(-1), b5a.reshape(-1)])[:, None].astype(f32)

    K = R0 + 9 * C1 + 25 * C3
    O5 = R0 + 9 * C1
    wm = jnp.zeros((Ctot, K), f32)
    wm = wm.at[:C0, :Cin].set(w1.T)
    wm = wm.at[C0 + C2 + C4:, Cin:R0].set(wp.T)
    for dh in range(3):
        for dw in range(3):
            wm = wm.at[C0:C0 + C2,
                       R0 + C1 * (3 * dh + dw):
                       R0 + C1 * (3 * dh + dw + 1)].set(w3b[dh, dw].T)
    for dh in range(5):
        for dw in range(5):
            wm = wm.at[C0 + C2:C0 + C2 + C4,
                       O5 + C3 * (5 * dh + dw):
                       O5 + C3 * (5 * dh + dw + 1)].set(w5b[dh, dw].T)
    wm = wm.astype(bf16)

    bo = jnp.concatenate([b1.reshape(-1), b3b.reshape(-1),
                          b5b.reshape(-1), bp.reshape(-1)])[:, None].astype(f32)

    x_flat = x.reshape(N, Cin, P)

    B = 4
    while N % B:
        B //= 2
    n_steps = N // B

    def vmem_spec(a):
        nd = a.ndim
        return pl.BlockSpec(a.shape, lambda _nd=nd: (0,) * _nd)

    weights = (wb, wm, bb, bo)
    out = pl.pallas_call(
        _make_kernel(H, W, Cin, C1, C3, C0, C2, C4, B, PAD, N // B),
        out_shape=jax.ShapeDtypeStruct((N, Ctot, P), f32),
        in_specs=[pl.BlockSpec(memory_space=pltpu.MemorySpace.HBM)]
                 + [vmem_spec(a) for a in weights],
        out_specs=pl.BlockSpec(memory_space=pltpu.MemorySpace.HBM),
        scratch_shapes=[pltpu.VMEM((2, B, Cin, P), f32),
                        pltpu.VMEM((2, B, Ctot, P), f32),
                        pltpu.VMEM((2, K, P), bf16),
                        pltpu.SemaphoreType.DMA((2,)),
                        pltpu.SemaphoreType.DMA((2,))],
    )(x_flat, *weights)

    return out.reshape(N, Ctot, H, W)


# speedup vs baseline: 1.0074x; 1.0074x over previous
import jax
import jax.numpy as jnp
from jax import lax
from jax.experimental import pallas as pl
from jax.experimental.pallas import tpu as pltpu


def _make_kernel(H, W, Cin, C1, C3, C0, C2, C4, B, PAD, n_steps):
    P = H * W
    Pp = P + 2 * PAD
    R0 = 2 * Cin
    R3B = 3 * C1
    R5B = 5 * C3
    O5 = R0 + 3 * R3B

    def compute_chunk(slot, x_buf, o_buf, wb_ref, wm_ref, bb_ref, bo_ref,
                      u_ref):
        lane = lax.broadcasted_iota(jnp.int32, (1, P), 1)
        wi = lane % W
        hi = lane // W
        def wshift(a, d, fill):
            if d == 0:
                return a
            m = (wi + d >= 0) & (wi + d < W)
            return jnp.where(m, pltpu.roll(a, (-d) % P, axis=1), fill)

        def hshift(a, d, fill):
            m = (hi + d >= 0) & (hi + d < H)
            return jnp.where(m, pltpu.roll(a, (-d * W) % P, axis=1), fill)

        for b in range(B):
            xb = x_buf[slot, b].astype(jnp.bfloat16)

            t = jnp.dot(wb_ref[...], xb, preferred_element_type=jnp.float32)
            t = (t + bb_ref[...]).astype(jnp.bfloat16)
            tp = jnp.pad(t, ((0, 0), (PAD, PAD)))
            tp3 = tp[:C1]
            tp5 = tp[C1:]

            neg = jnp.array(-jnp.inf, jnp.bfloat16)
            m1 = jnp.maximum(jnp.maximum(wshift(xb, -1, neg),
                                         wshift(xb, 1, neg)), xb)
            mp = jnp.maximum(jnp.maximum(hshift(m1, -1, neg),
                                         hshift(m1, 1, neg)), m1)

            p = b % 2
            u_ref[p, 0:Cin] = xb
            u_ref[p, Cin:R0] = mp
            zero = jnp.zeros((), jnp.bfloat16)
            for j, dh in enumerate((-1, 0, 1)):
                for i, dw in enumerate((-1, 0, 1)):
                    m = (wi + dw >= 0) & (wi + dw < W)
                    r = pltpu.roll(tp3, (-(dh * W + dw)) % Pp,
                                   axis=1)[:, PAD:PAD + P]
                    u_ref[p, R0 + C1 * (3 * j + i):
                          R0 + C1 * (3 * j + i + 1)] = jnp.where(m, r, zero)
            for j, dh in enumerate((-2, -1, 0, 1, 2)):
                for i, dw in enumerate((-2, -1, 0, 1, 2)):
                    m = (wi + dw >= 0) & (wi + dw < W)
                    r = pltpu.roll(tp5, (-(dh * W + dw)) % Pp,
                                   axis=1)[:, PAD:PAD + P]
                    u_ref[p, O5 + C3 * (5 * j + i):
                          O5 + C3 * (5 * j + i + 1)] = jnp.where(m, r, zero)

            HP = P // 2
            for h in range(2):
                acc = jnp.dot(wm_ref[...], u_ref[p, :, h * HP:(h + 1) * HP],
                              preferred_element_type=jnp.float32)
                o_buf[slot, b, :, h * HP:(h + 1) * HP] = acc + bo_ref[...]

    def body(x_hbm, wb_ref, wm_ref, bb_ref, bo_ref, o_hbm,
             x_buf, o_buf, u_ref, in_sem, out_sem):

        def dma_in(slot, step):
            pltpu.make_async_copy(x_hbm.at[pl.ds(step * B, B)],
                                  x_buf.at[slot], in_sem.at[slot]).start()

        def wait_in(slot):
            pltpu.make_async_copy(x_hbm.at[pl.ds(0, B)],
                                  x_buf.at[slot], in_sem.at[slot]).wait()

        def dma_out(slot, step):
            for b in range(B):
                pltpu.make_async_copy(o_buf.at[slot, b],
                                      o_hbm.at[step * B + b],
                                      out_sem.at[slot]).start()

        def wait_out(slot):
            for b in range(B):
                pltpu.make_async_copy(o_buf.at[slot, 0],
                                      o_hbm.at[0],
                                      out_sem.at[slot]).wait()

        dma_in(0, 0)

        def step_body(step, _):
            cur = lax.rem(step, 2)
            nxt = lax.rem(step + 1, 2)

            @pl.when(step + 1 < n_steps)
            def _():
                dma_in(nxt, step + 1)

            wait_in(cur)

            @pl.when(step >= 2)
            def _():
                wait_out(cur)

            compute_chunk(cur, x_buf, o_buf, wb_ref, wm_ref, bb_ref, bo_ref,
                          u_ref)
            dma_out(cur, step)
            return ()

        lax.fori_loop(0, n_steps, step_body, ())
        if n_steps >= 2:
            wait_out(jnp.int32((n_steps - 2) % 2))
        wait_out(jnp.int32((n_steps - 1) % 2))

    return body


def kernel(x, w1, b1, w3a, b3a, w3b, b3b, w5a, b5a, w5b, b5b, wp, bp):
    N, Cin, H, W = x.shape
    C0 = w1.shape[1]
    C1, C2 = w3b.shape[2], w3b.shape[3]
    C3, C4 = w5b.shape[2], w5b.shape[3]
    C5 = wp.shape[1]
    Ctot = C0 + C2 + C4 + C5
    P = H * W
    PAD = 128
    R0 = 2 * Cin

    f32, bf16 = jnp.float32, jnp.bfloat16

    wb = jnp.concatenate([w3a.T, w5a.T], axis=0).astype(bf16)
    bb = jnp.concatenate([b3a.reshape(-1), b5a.reshape(-1)])[:, None].astype(f32)

    K = R0 + 9 * C1 + 25 * C3
    O5 = R0 + 9 * C1
    wm = jnp.zeros((Ctot, K), f32)
    wm = wm.at[:C0, :Cin].set(w1.T)
    wm = wm.at[C0 + C2 + C4:, Cin:R0].set(wp.T)
    for dh in range(3):
        for dw in range(3):
            wm = wm.at[C0:C0 + C2,
                       R0 + C1 * (3 * dh + dw):
                       R0 + C1 * (3 * dh + dw + 1)].set(w3b[dh, dw].T)
    for dh in range(5):
        for dw in range(5):
            wm = wm.at[C0 + C2:C0 + C2 + C4,
                       O5 + C3 * (5 * dh + dw):
                       O5 + C3 * (5 * dh + dw + 1)].set(w5b[dh, dw].T)
    wm = wm.astype(bf16)

    bo = jnp.concatenate([b1.reshape(-1), b3b.reshape(-1),
                          b5b.reshape(-1), bp.reshape(-1)])[:, None].astype(f32)

    x_flat = x.reshape(N, Cin, P)

    B = 8
    while N % B:
        B //= 2
    n_steps = N // B

    def vmem_spec(a):
        nd = a.ndim
        return pl.BlockSpec(a.shape, lambda _nd=nd: (0,) * _nd)

    weights = (wb, wm, bb, bo)
    out = pl.pallas_call(
        _make_kernel(H, W, Cin, C1, C3, C0, C2, C4, B, PAD, N // B),
        out_shape=jax.ShapeDtypeStruct((N, Ctot, P), f32),
        in_specs=[pl.BlockSpec(memory_space=pltpu.MemorySpace.HBM)]
                 + [vmem_spec(a) for a in weights],
        out_specs=pl.BlockSpec(memory_space=pltpu.MemorySpace.HBM),
        scratch_shapes=[pltpu.VMEM((2, B, Cin, P), f32),
                        pltpu.VMEM((2, B, Ctot, P), f32),
                        pltpu.VMEM((2, K, P), bf16),
                        pltpu.SemaphoreType.DMA((2,)),
                        pltpu.SemaphoreType.DMA((2,))],
    )(x_flat, *weights)

    return out.reshape(N, Ctot, H, W)


# speedup vs baseline: 1.0266x; 1.0191x over previous
import jax
import jax.numpy as jnp
from jax import lax
from jax.experimental import pallas as pl
from jax.experimental.pallas import tpu as pltpu


def _make_kernel(H, W, Cin, C1, C3, C0, C2, C4, B, PAD, n_steps):
    P = H * W
    Pp = P + 2 * PAD
    R0 = 2 * Cin
    R3B = 3 * C1
    R5B = 5 * C3
    O5 = R0 + 3 * R3B

    def compute_chunk(slot, step, x_buf, o_buf, o_hbm, out_sem,
                      wb_ref, wm_ref, bb_ref, bo_ref, u_ref):
        lane = lax.broadcasted_iota(jnp.int32, (1, P), 1)
        wi = lane % W
        hi = lane // W
        def wshift(a, d, fill):
            if d == 0:
                return a
            m = (wi + d >= 0) & (wi + d < W)
            return jnp.where(m, pltpu.roll(a, (-d) % P, axis=1), fill)

        def hshift(a, d, fill):
            m = (hi + d >= 0) & (hi + d < H)
            return jnp.where(m, pltpu.roll(a, (-d * W) % P, axis=1), fill)

        for b in range(B):
            xb = x_buf[slot, b].astype(jnp.bfloat16)

            t = jnp.dot(wb_ref[...], xb, preferred_element_type=jnp.float32)
            t = (t + bb_ref[...]).astype(jnp.bfloat16)
            tp = jnp.pad(t, ((0, 0), (PAD, PAD)))
            tp3 = tp[:C1]
            tp5 = tp[C1:]

            neg = jnp.array(-jnp.inf, jnp.bfloat16)
            m1 = jnp.maximum(jnp.maximum(wshift(xb, -1, neg),
                                         wshift(xb, 1, neg)), xb)
            mp = jnp.maximum(jnp.maximum(hshift(m1, -1, neg),
                                         hshift(m1, 1, neg)), m1)

            p = b % 2
            u_ref[p, 0:Cin] = xb
            u_ref[p, Cin:R0] = mp
            zero = jnp.zeros((), jnp.bfloat16)
            for j, dh in enumerate((-1, 0, 1)):
                for i, dw in enumerate((-1, 0, 1)):
                    m = (wi + dw >= 0) & (wi + dw < W)
                    r = pltpu.roll(tp3, (-(dh * W + dw)) % Pp,
                                   axis=1)[:, PAD:PAD + P]
                    u_ref[p, R0 + C1 * (3 * j + i):
                          R0 + C1 * (3 * j + i + 1)] = jnp.where(m, r, zero)
            for j, dh in enumerate((-2, -1, 0, 1, 2)):
                for i, dw in enumerate((-2, -1, 0, 1, 2)):
                    m = (wi + dw >= 0) & (wi + dw < W)
                    r = pltpu.roll(tp5, (-(dh * W + dw)) % Pp,
                                   axis=1)[:, PAD:PAD + P]
                    u_ref[p, O5 + C3 * (5 * j + i):
                          O5 + C3 * (5 * j + i + 1)] = jnp.where(m, r, zero)

            acc = jnp.dot(wm_ref[...], u_ref[p],
                          preferred_element_type=jnp.float32)
            o_buf[slot, b] = acc + bo_ref[...]
            pltpu.make_async_copy(o_buf.at[slot, b],
                                  o_hbm.at[step * B + b],
                                  out_sem.at[slot]).start()

    def body(x_hbm, wb_ref, wm_ref, bb_ref, bo_ref, o_hbm,
             x_buf, o_buf, u_ref, in_sem, out_sem):

        def dma_in(slot, step):
            pltpu.make_async_copy(x_hbm.at[pl.ds(step * B, B)],
                                  x_buf.at[slot], in_sem.at[slot]).start()

        def wait_in(slot):
            pltpu.make_async_copy(x_hbm.at[pl.ds(0, B)],
                                  x_buf.at[slot], in_sem.at[slot]).wait()

        def wait_out(slot):
            for b in range(B):
                pltpu.make_async_copy(o_buf.at[slot, 0],
                                      o_hbm.at[0],
                                      out_sem.at[slot]).wait()

        dma_in(0, 0)

        def step_body(step, _):
            cur = lax.rem(step, 2)
            nxt = lax.rem(step + 1, 2)

            @pl.when(step + 1 < n_steps)
            def _():
                dma_in(nxt, step + 1)

            wait_in(cur)

            @pl.when(step >= 2)
            def _():
                wait_out(cur)

            compute_chunk(cur, step, x_buf, o_buf, o_hbm, out_sem,
                          wb_ref, wm_ref, bb_ref, bo_ref, u_ref)
            return ()

        lax.fori_loop(0, n_steps, step_body, ())
        if n_steps >= 2:
            wait_out(jnp.int32((n_steps - 2) % 2))
        wait_out(jnp.int32((n_steps - 1) % 2))

    return body


def kernel(x, w1, b1, w3a, b3a, w3b, b3b, w5a, b5a, w5b, b5b, wp, bp):
    N, Cin, H, W = x.shape
    C0 = w1.shape[1]
    C1, C2 = w3b.shape[2], w3b.shape[3]
    C3, C4 = w5b.shape[2], w5b.shape[3]
    C5 = wp.shape[1]
    Ctot = C0 + C2 + C4 + C5
    P = H * W
    PAD = 128
    R0 = 2 * Cin

    f32, bf16 = jnp.float32, jnp.bfloat16

    wb = jnp.concatenate([w3a.T, w5a.T], axis=0).astype(bf16)
    bb = jnp.concatenate([b3a.reshape(-1), b5a.reshape(-1)])[:, None].astype(f32)

    K = R0 + 9 * C1 + 25 * C3
    O5 = R0 + 9 * C1
    wm = jnp.zeros((Ctot, K), f32)
    wm = wm.at[:C0, :Cin].set(w1.T)
    wm = wm.at[C0 + C2 + C4:, Cin:R0].set(wp.T)
    for dh in range(3):
        for dw in range(3):
            wm = wm.at[C0:C0 + C2,
                       R0 + C1 * (3 * dh + dw):
                       R0 + C1 * (3 * dh + dw + 1)].set(w3b[dh, dw].T)
    for dh in range(5):
        for dw in range(5):
            wm = wm.at[C0 + C2:C0 + C2 + C4,
                       O5 + C3 * (5 * dh + dw):
                       O5 + C3 * (5 * dh + dw + 1)].set(w5b[dh, dw].T)
    wm = wm.astype(bf16)

    bo = jnp.concatenate([b1.reshape(-1), b3b.reshape(-1),
                          b5b.reshape(-1), bp.reshape(-1)])[:, None].astype(f32)

    x_flat = x.reshape(N, Cin, P)

    B = 8
    while N % B:
        B //= 2
    n_steps = N // B

    def vmem_spec(a):
        nd = a.ndim
        return pl.BlockSpec(a.shape, lambda _nd=nd: (0,) * _nd)

    weights = (wb, wm, bb, bo)
    out = pl.pallas_call(
        _make_kernel(H, W, Cin, C1, C3, C0, C2, C4, B, PAD, N // B),
        out_shape=jax.ShapeDtypeStruct((N, Ctot, P), f32),
        in_specs=[pl.BlockSpec(memory_space=pltpu.MemorySpace.HBM)]
                 + [vmem_spec(a) for a in weights],
        out_specs=pl.BlockSpec(memory_space=pltpu.MemorySpace.HBM),
        scratch_shapes=[pltpu.VMEM((2, B, Cin, P), f32),
                        pltpu.VMEM((2, B, Ctot, P), f32),
                        pltpu.VMEM((2, K, P), bf16),
                        pltpu.SemaphoreType.DMA((2,)),
                        pltpu.SemaphoreType.DMA((2,))],
    )(x_flat, *weights)

    return out.reshape(N, Ctot, H, W)


# speedup vs baseline: 1.0399x; 1.0129x over previous
import jax
import jax.numpy as jnp
from jax import lax
from jax.experimental import pallas as pl
from jax.experimental.pallas import tpu as pltpu


def _make_kernel(H, W, Cin, C1, C3, C0, C2, C4, B, PAD, n_steps):
    P = H * W
    Pp = P + 2 * PAD
    R0 = 2 * Cin
    R3B = 3 * C1
    R5B = 5 * C3
    O5 = R0 + 3 * R3B

    def compute_chunk(slot, step, x_buf, o_buf, o_hbm, out_sem,
                      wb_ref, wm_ref, bb_ref, bo_ref, u_ref):
        lane = lax.broadcasted_iota(jnp.int32, (1, P), 1)
        wi = lane % W
        hi = lane // W
        def wshift(a, d, fill):
            if d == 0:
                return a
            m = (wi + d >= 0) & (wi + d < W)
            return jnp.where(m, pltpu.roll(a, (-d) % P, axis=1), fill)

        def hshift(a, d, fill):
            m = (hi + d >= 0) & (hi + d < H)
            return jnp.where(m, pltpu.roll(a, (-d * W) % P, axis=1), fill)

        for b in range(B):
            xb = x_buf[slot, b].astype(jnp.bfloat16)

            t = jnp.dot(wb_ref[...], xb, preferred_element_type=jnp.float32)
            t = (t + bb_ref[...]).astype(jnp.bfloat16)
            tp = jnp.pad(t, ((0, 0), (PAD, PAD)))
            tp3 = tp[:C1]
            tp5 = tp[C1:]

            neg = jnp.array(-jnp.inf, jnp.bfloat16)
            m1 = jnp.maximum(jnp.maximum(wshift(xb, -1, neg),
                                         wshift(xb, 1, neg)), xb)
            mp = jnp.maximum(jnp.maximum(hshift(m1, -1, neg),
                                         hshift(m1, 1, neg)), m1)

            p = b % 2
            u_ref[p, 0:Cin] = xb
            u_ref[p, Cin:R0] = mp
            zero = jnp.zeros((), jnp.bfloat16)
            for j, dh in enumerate((-1, 0, 1)):
                for i, dw in enumerate((-1, 0, 1)):
                    m = (wi + dw >= 0) & (wi + dw < W)
                    r = pltpu.roll(tp3, (-(dh * W + dw)) % Pp,
                                   axis=1)[:, PAD:PAD + P]
                    u_ref[p, R0 + C1 * (3 * j + i):
                          R0 + C1 * (3 * j + i + 1)] = jnp.where(m, r, zero)
            for j, dh in enumerate((-2, -1, 0, 1, 2)):
                for i, dw in enumerate((-2, -1, 0, 1, 2)):
                    m = (wi + dw >= 0) & (wi + dw < W)
                    r = pltpu.roll(tp5, (-(dh * W + dw)) % Pp,
                                   axis=1)[:, PAD:PAD + P]
                    u_ref[p, O5 + C3 * (5 * j + i):
                          O5 + C3 * (5 * j + i + 1)] = jnp.where(m, r, zero)

            acc = jnp.dot(wm_ref[...], u_ref[p],
                          preferred_element_type=jnp.float32)
            o_buf[slot, b] = acc + bo_ref[...]
            pltpu.make_async_copy(o_buf.at[slot, b],
                                  o_hbm.at[step * B + b],
                                  out_sem.at[slot]).start()

    def body(x_hbm, wb_ref, wm_ref, bb_ref, bo_ref, o_hbm,
             x_buf, o_buf, u_ref, in_sem, out_sem):

        def dma_in(slot, step):
            pltpu.make_async_copy(x_hbm.at[pl.ds(step * B, B)],
                                  x_buf.at[slot], in_sem.at[slot]).start()

        def wait_in(slot):
            pltpu.make_async_copy(x_hbm.at[pl.ds(0, B)],
                                  x_buf.at[slot], in_sem.at[slot]).wait()

        def wait_out(slot):
            for b in range(B):
                pltpu.make_async_copy(o_buf.at[slot, 0],
                                      o_hbm.at[0],
                                      out_sem.at[slot]).wait()

        dma_in(0, 0)

        def step_body(step, _):
            cur = lax.rem(step, 2)
            nxt = lax.rem(step + 1, 2)

            @pl.when(step + 1 < n_steps)
            def _():
                dma_in(nxt, step + 1)

            wait_in(cur)

            @pl.when(step >= 2)
            def _():
                wait_out(cur)

            compute_chunk(cur, step, x_buf, o_buf, o_hbm, out_sem,
                          wb_ref, wm_ref, bb_ref, bo_ref, u_ref)
            return ()

        lax.fori_loop(0, n_steps, step_body, ())
        if n_steps >= 2:
            wait_out(jnp.int32((n_steps - 2) % 2))
        wait_out(jnp.int32((n_steps - 1) % 2))

    return body


def kernel(x, w1, b1, w3a, b3a, w3b, b3b, w5a, b5a, w5b, b5b, wp, bp):
    N, Cin, H, W = x.shape
    C0 = w1.shape[1]
    C1, C2 = w3b.shape[2], w3b.shape[3]
    C3, C4 = w5b.shape[2], w5b.shape[3]
    C5 = wp.shape[1]
    Ctot = C0 + C2 + C4 + C5
    P = H * W
    PAD = 128
    R0 = 2 * Cin

    f32, bf16 = jnp.float32, jnp.bfloat16

    wb = jnp.concatenate([w3a.T, w5a.T], axis=0).astype(bf16)
    bb = jnp.concatenate([b3a.reshape(-1), b5a.reshape(-1)])[:, None].astype(f32)

    K = R0 + 9 * C1 + 25 * C3
    O5 = R0 + 9 * C1
    wm = jnp.zeros((Ctot, K), f32)
    wm = wm.at[:C0, :Cin].set(w1.T)
    wm = wm.at[C0 + C2 + C4:, Cin:R0].set(wp.T)
    for dh in range(3):
        for dw in range(3):
            wm = wm.at[C0:C0 + C2,
                       R0 + C1 * (3 * dh + dw):
                       R0 + C1 * (3 * dh + dw + 1)].set(w3b[dh, dw].T)
    for dh in range(5):
        for dw in range(5):
            wm = wm.at[C0 + C2:C0 + C2 + C4,
                       O5 + C3 * (5 * dh + dw):
                       O5 + C3 * (5 * dh + dw + 1)].set(w5b[dh, dw].T)
    wm = wm.astype(bf16)

    bo = jnp.concatenate([b1.reshape(-1), b3b.reshape(-1),
                          b5b.reshape(-1), bp.reshape(-1)])[:, None].astype(f32)

    x_flat = x.reshape(N, Cin, P)

    B = 16
    while N % B:
        B //= 2
    n_steps = N // B

    def vmem_spec(a):
        nd = a.ndim
        return pl.BlockSpec(a.shape, lambda _nd=nd: (0,) * _nd)

    weights = (wb, wm, bb, bo)
    out = pl.pallas_call(
        _make_kernel(H, W, Cin, C1, C3, C0, C2, C4, B, PAD, N // B),
        out_shape=jax.ShapeDtypeStruct((N, Ctot, P), f32),
        in_specs=[pl.BlockSpec(memory_space=pltpu.MemorySpace.HBM)]
                 + [vmem_spec(a) for a in weights],
        out_specs=pl.BlockSpec(memory_space=pltpu.MemorySpace.HBM),
        scratch_shapes=[pltpu.VMEM((2, B, Cin, P), f32),
                        pltpu.VMEM((2, B, Ctot, P), f32),
                        pltpu.VMEM((2, K, P), bf16),
                        pltpu.SemaphoreType.DMA((2,)),
                        pltpu.SemaphoreType.DMA((2,))],
    )(x_flat, *weights)

    return out.reshape(N, Ctot, H, W)


# speedup vs baseline: 1.0452x; 1.0051x over previous
import jax
import jax.numpy as jnp
from jax import lax
from jax.experimental import pallas as pl
from jax.experimental.pallas import tpu as pltpu


def _make_kernel(H, W, Cin, C1, C3, C0, C2, C4, B, PAD, n_steps):
    P = H * W
    Pp = P + 2 * PAD
    R0 = 2 * Cin
    R3B = 3 * C1
    R5B = 5 * C3
    O5 = R0 + 3 * R3B

    def compute_chunk(slot, step, x_buf, o_buf, o_hbm, out_sem,
                      wb_ref, wm_ref, bb_ref, bo_ref, u_ref):
        lane = lax.broadcasted_iota(jnp.int32, (1, P), 1)
        wi = lane % W
        hi = lane // W
        def wshift(a, d, fill):
            if d == 0:
                return a
            m = (wi + d >= 0) & (wi + d < W)
            return jnp.where(m, pltpu.roll(a, (-d) % P, axis=1), fill)

        def hshift(a, d, fill):
            m = (hi + d >= 0) & (hi + d < H)
            return jnp.where(m, pltpu.roll(a, (-d * W) % P, axis=1), fill)

        for b in range(B):
            xb = x_buf[slot, b].astype(jnp.bfloat16)

            t = jnp.dot(wb_ref[...], xb, preferred_element_type=jnp.float32)
            t = (t + bb_ref[...]).astype(jnp.bfloat16)
            tp = jnp.pad(t, ((0, 0), (PAD, PAD)))
            tp3 = tp[:C1]
            tp5 = tp[C1:]

            neg = jnp.array(-jnp.inf, jnp.bfloat16)
            m1 = jnp.maximum(jnp.maximum(wshift(xb, -1, neg),
                                         wshift(xb, 1, neg)), xb)
            mp = jnp.maximum(jnp.maximum(hshift(m1, -1, neg),
                                         hshift(m1, 1, neg)), m1)

            p = b % 2
            u_ref[p, 0:Cin] = xb
            u_ref[p, Cin:R0] = mp
            zero = jnp.zeros((), jnp.bfloat16)
            for j, dh in enumerate((-1, 0, 1)):
                for i, dw in enumerate((-1, 0, 1)):
                    m = (wi + dw >= 0) & (wi + dw < W)
                    r = pltpu.roll(tp3, (-(dh * W + dw)) % Pp,
                                   axis=1)[:, PAD:PAD + P]
                    u_ref[p, R0 + C1 * (3 * j + i):
                          R0 + C1 * (3 * j + i + 1)] = jnp.where(m, r, zero)
            for j, dh in enumerate((-2, -1, 0, 1, 2)):
                for i, dw in enumerate((-2, -1, 0, 1, 2)):
                    m = (wi + dw >= 0) & (wi + dw < W)
                    r = pltpu.roll(tp5, (-(dh * W + dw)) % Pp,
                                   axis=1)[:, PAD:PAD + P]
                    u_ref[p, O5 + C3 * (5 * j + i):
                          O5 + C3 * (5 * j + i + 1)] = jnp.where(m, r, zero)

            acc = jnp.dot(wm_ref[...], u_ref[p],
                          preferred_element_type=jnp.float32)
            o_buf[slot, b] = acc + bo_ref[...]
            pltpu.make_async_copy(o_buf.at[slot, b],
                                  o_hbm.at[step * B + b],
                                  out_sem.at[slot]).start()

    def body(x_hbm, wb_ref, wm_ref, bb_ref, bo_ref, o_hbm,
             x_buf, o_buf, u_ref, in_sem, out_sem):

        def dma_in(slot, step):
            pltpu.make_async_copy(x_hbm.at[pl.ds(step * B, B)],
                                  x_buf.at[slot], in_sem.at[slot]).start()

        def wait_in(slot):
            pltpu.make_async_copy(x_hbm.at[pl.ds(0, B)],
                                  x_buf.at[slot], in_sem.at[slot]).wait()

        def wait_out(slot):
            for b in range(B):
                pltpu.make_async_copy(o_buf.at[slot, 0],
                                      o_hbm.at[0],
                                      out_sem.at[slot]).wait()

        dma_in(0, 0)

        def step_body(step, _):
            cur = lax.rem(step, 2)
            nxt = lax.rem(step + 1, 2)

            @pl.when(step + 1 < n_steps)
            def _():
                dma_in(nxt, step + 1)

            wait_in(cur)

            @pl.when(step >= 2)
            def _():
                wait_out(cur)

            compute_chunk(cur, step, x_buf, o_buf, o_hbm, out_sem,
                          wb_ref, wm_ref, bb_ref, bo_ref, u_ref)
            return ()

        lax.fori_loop(0, n_steps, step_body, ())
        if n_steps >= 2:
            wait_out(jnp.int32((n_steps - 2) % 2))
        wait_out(jnp.int32((n_steps - 1) % 2))

    return body


def kernel(x, w1, b1, w3a, b3a, w3b, b3b, w5a, b5a, w5b, b5b, wp, bp):
    N, Cin, H, W = x.shape
    C0 = w1.shape[1]
    C1, C2 = w3b.shape[2], w3b.shape[3]
    C3, C4 = w5b.shape[2], w5b.shape[3]
    C5 = wp.shape[1]
    Ctot = C0 + C2 + C4 + C5
    P = H * W
    PAD = 128
    R0 = 2 * Cin

    f32, bf16 = jnp.float32, jnp.bfloat16

    wb = jnp.concatenate([w3a.T, w5a.T], axis=0).astype(bf16)
    bb = jnp.concatenate([b3a.reshape(-1), b5a.reshape(-1)])[:, None].astype(f32)

    K = R0 + 9 * C1 + 25 * C3
    O5 = R0 + 9 * C1
    wm = jnp.zeros((Ctot, K), f32)
    wm = wm.at[:C0, :Cin].set(w1.T)
    wm = wm.at[C0 + C2 + C4:, Cin:R0].set(wp.T)
    for dh in range(3):
        for dw in range(3):
            wm = wm.at[C0:C0 + C2,
                       R0 + C1 * (3 * dh + dw):
                       R0 + C1 * (3 * dh + dw + 1)].set(w3b[dh, dw].T)
    for dh in range(5):
        for dw in range(5):
            wm = wm.at[C0 + C2:C0 + C2 + C4,
                       O5 + C3 * (5 * dh + dw):
                       O5 + C3 * (5 * dh + dw + 1)].set(w5b[dh, dw].T)
    wm = wm.astype(bf16)

    bo = jnp.concatenate([b1.reshape(-1), b3b.reshape(-1),
                          b5b.reshape(-1), bp.reshape(-1)])[:, None].astype(f32)

    x_flat = x.reshape(N, Cin, P)

    B = 32
    while N % B:
        B //= 2
    n_steps = N // B

    def vmem_spec(a):
        nd = a.ndim
        return pl.BlockSpec(a.shape, lambda _nd=nd: (0,) * _nd)

    weights = (wb, wm, bb, bo)
    out = pl.pallas_call(
        _make_kernel(H, W, Cin, C1, C3, C0, C2, C4, B, PAD, N // B),
        out_shape=jax.ShapeDtypeStruct((N, Ctot, P), f32),
        in_specs=[pl.BlockSpec(memory_space=pltpu.MemorySpace.HBM)]
                 + [vmem_spec(a) for a in weights],
        out_specs=pl.BlockSpec(memory_space=pltpu.MemorySpace.HBM),
        scratch_shapes=[pltpu.VMEM((2, B, Cin, P), f32),
                        pltpu.VMEM((2, B, Ctot, P), f32),
                        pltpu.VMEM((2, K, P), bf16),
                        pltpu.SemaphoreType.DMA((2,)),
                        pltpu.SemaphoreType.DMA((2,))],
    )(x_flat, *weights)

    return out.reshape(N, Ctot, H, W)
